# Initial kernel scaffold; baseline (speedup 1.0000x reference)
import jax
import jax.numpy as jnp
from jax.experimental import pallas as pl


def kernel(x, h, mask, edge_index, W_in, b_in):
    raise NotImplementedError("write your pallas kernel here")


# speedup vs baseline: 7.4071x; 7.4071x over previous
import functools

import jax
import jax.numpy as jnp
from jax import lax
from jax.experimental import pallas as pl
from jax.experimental.pallas import tpu as pltpu
from jax.experimental.pallas import tpu_sc as plsc

N = 10000
E = 320000
D = 128

NC = 2
NS = 16
NW = NC * NS
EPW = E // NW
CHUNK = 80
NCHUNK = EPW // CHUNK
NPAD = 10240
RPS = NPAD // NS


def _sc_segment_sum(h, src, dst, zeros):
  mesh = plsc.VectorSubcoreMesh(core_axis_name="c", subcore_axis_name="s")

  @functools.partial(
      pl.kernel,
      out_type=jax.ShapeDtypeStruct((NC, NPAD, D), jnp.float32),
      mesh=mesh,
      scratch_types=[
          pltpu.VMEM((NCHUNK, CHUNK), jnp.int32),
          pltpu.VMEM((NCHUNK, CHUNK), jnp.int32),
          pltpu.VMEM((CHUNK, D), jnp.float32),
          pltpu.VMEM_SHARED((NPAD, D), jnp.float32),
          pltpu.SemaphoreType.DMA,
      ],
  )
  def seg_sum(h_hbm, src_hbm, dst_hbm, z_hbm, out_hbm,
              src_v, dst_v, rows_v, acc, sem):
    c = lax.axis_index("c")
    s = lax.axis_index("s")
    wid = c * NS + s
    rbase = s * RPS
    pltpu.sync_copy(z_hbm.at[pl.ds(rbase, RPS)], acc.at[pl.ds(rbase, RPS)])
    pltpu.sync_copy(src_hbm.at[wid], src_v)
    pltpu.sync_copy(dst_hbm.at[wid], dst_v)
    plsc.subcore_barrier()

    def body(i, carry):
      pltpu.async_copy(h_hbm.at[src_v.at[i]], rows_v, sem).wait()
      pltpu.sync_copy(rows_v, acc.at[dst_v.at[i]], add=True)
      return carry

    lax.fori_loop(0, NCHUNK, body, 0)
    plsc.subcore_barrier()
    pltpu.sync_copy(acc.at[pl.ds(rbase, RPS)],
                    out_hbm.at[c, pl.ds(rbase, RPS)])

  return seg_sum(h, src, dst, zeros)


def _tc_combine(x, maskf, W, b, p0, p1):
  BLK = 2000

  def body(x_ref, m_ref, w_ref, b_ref, p0_ref, p1_ref, o_ref):
    hin = (lax.dot_general(x_ref[...], w_ref[...], (((1,), (1,)), ((), ())),
                           preferred_element_type=jnp.float32)
           + b_ref[...]) * m_ref[...]
    o_ref[...] = jnp.tanh(hin + p0_ref[...] + p1_ref[...])

  return pl.pallas_call(
      body,
      grid=(N // BLK,),
      in_specs=[
          pl.BlockSpec((BLK, D), lambda i: (i, 0)),
          pl.BlockSpec((BLK, 1), lambda i: (i, 0)),
          pl.BlockSpec((D, D), lambda i: (0, 0)),
          pl.BlockSpec((1, D), lambda i: (0, 0)),
          pl.BlockSpec((BLK, D), lambda i: (i, 0)),
          pl.BlockSpec((BLK, D), lambda i: (i, 0)),
      ],
      out_specs=pl.BlockSpec((BLK, D), lambda i: (i, 0)),
      out_shape=jax.ShapeDtypeStruct((N, D), jnp.float32),
  )(x, maskf, W, b, p0, p1)


def kernel(x, h, mask, edge_index, W_in, b_in):
  src = edge_index[0].astype(jnp.int32).reshape(NW, NCHUNK, CHUNK)
  dst = edge_index[1].astype(jnp.int32).reshape(NW, NCHUNK, CHUNK)
  zeros = jnp.zeros((NPAD, D), jnp.float32)
  partials = _sc_segment_sum(h, src, dst, zeros)
  maskf = mask.astype(jnp.float32)[:, None]
  return _tc_combine(x, maskf, W_in, b_in.reshape(1, D),
                     partials[0, :N], partials[1, :N])


# speedup vs baseline: 11.9845x; 1.6180x over previous
import functools

import jax
import jax.numpy as jnp
from jax import lax
from jax.experimental import pallas as pl
from jax.experimental.pallas import tpu as pltpu
from jax.experimental.pallas import tpu_sc as plsc

N = 10000
E = 320000
D = 128

NC = 2
NS = 16
NW = NC * NS
EPW = E // NW
CHUNK = 128
NCHUNK = 80
EPP = NCHUNK * CHUNK
PADW = EPP - EPW
NPAD = 10240
RPS = NPAD // NS


def _sc_segment_sum(h, src, dst, zeros):
  mesh = plsc.VectorSubcoreMesh(core_axis_name="c", subcore_axis_name="s")

  @functools.partial(
      pl.kernel,
      out_type=jax.ShapeDtypeStruct((NC, NPAD, D), jnp.float32),
      mesh=mesh,
      scratch_types=[
          pltpu.VMEM_SHARED((NPAD, D), jnp.float32),
          pltpu.VMEM((CHUNK, D), jnp.float32),
          pltpu.VMEM((CHUNK, D), jnp.float32),
          pltpu.VMEM((CHUNK,), jnp.int32),
          pltpu.VMEM((CHUNK,), jnp.int32),
          pltpu.VMEM((CHUNK,), jnp.int32),
          pltpu.VMEM((CHUNK,), jnp.int32),
          pltpu.VMEM((CHUNK,), jnp.int32),
          pltpu.VMEM((CHUNK,), jnp.int32),
          pltpu.VMEM((CHUNK,), jnp.int32),
          pltpu.VMEM((CHUNK,), jnp.int32),
          pltpu.SemaphoreType.DMA,
          pltpu.SemaphoreType.DMA,
          pltpu.SemaphoreType.DMA,
          pltpu.SemaphoreType.DMA,
          pltpu.SemaphoreType.DMA,
          pltpu.SemaphoreType.DMA,
          pltpu.SemaphoreType.DMA,
          pltpu.SemaphoreType.DMA,
          pltpu.SemaphoreType.DMA,
          pltpu.SemaphoreType.DMA,
      ],
  )
  def seg_sum(h_hbm, src_hbm, dst_hbm, z_hbm, out_hbm, acc,
              rows0, rows1, ss0, ss1, ss2, ss3, ds0, ds1, ds2, ds3,
              g0, g1, sm0, sm1, sm2, sm3, dm0, dm1, dm2, dm3):
    rows = (rows0, rows1)
    sslot = (ss0, ss1, ss2, ss3)
    dslot = (ds0, ds1, ds2, ds3)
    gsem = (g0, g1)
    ssem = (sm0, sm1, sm2, sm3)
    dsem = (dm0, dm1, dm2, dm3)

    c = lax.axis_index("c")
    s = lax.axis_index("s")
    wid = c * NS + s
    ebase = wid * EPP
    rbase = s * RPS

    def fetch_src(i, q):
      pltpu.async_copy(src_hbm.at[pl.ds(ebase + i * CHUNK, CHUNK)],
                       sslot[q], ssem[q])

    def fetch_dst(i, q):
      pltpu.async_copy(dst_hbm.at[pl.ds(ebase + i * CHUNK, CHUNK)],
                       dslot[q], dsem[q])

    def wait_idx(slot, sem):
      pltpu.make_async_copy(src_hbm.at[pl.ds(ebase, CHUNK)], slot, sem).wait()

    def gather(q, b):
      pltpu.async_copy(h_hbm.at[sslot[q]], rows[b], gsem[b])

    def wait_gather(b):
      pltpu.make_async_copy(h_hbm.at[sslot[0]], rows[b], gsem[b]).wait()

    pltpu.sync_copy(z_hbm.at[pl.ds(rbase, RPS)], acc.at[pl.ds(rbase, RPS)])
    for q in range(4):
      fetch_src(q, q)
      fetch_dst(q, q)
    plsc.subcore_barrier()
    for b in range(2):
      wait_idx(sslot[b], ssem[b])
      gather(b, b)

    def step(i, b, q, do_fetch, gather_i2):
      wait_gather(b)
      if do_fetch:
        fetch_src(i + 4, q)
      wait_idx(dslot[q], dsem[q])
      pltpu.sync_copy(rows[b], acc.at[dslot[q]], add=True)
      if do_fetch:
        fetch_dst(i + 4, q)
      if gather_i2:
        wait_idx(sslot[(q + 2) % 4], ssem[(q + 2) % 4])
        gather((q + 2) % 4, b)

    def outer(io, carry):
      for k in range(4):
        i = 4 * io + k
        step(i, k % 2, k, True, True)
      return carry

    lax.fori_loop(0, (NCHUNK - 4) // 4, outer, 0)
    i0 = NCHUNK - 4
    step(i0 + 0, 0, 0, False, True)
    step(i0 + 1, 1, 1, False, True)
    step(i0 + 2, 0, 2, False, False)
    step(i0 + 3, 1, 3, False, False)

    plsc.subcore_barrier()
    pltpu.sync_copy(acc.at[pl.ds(rbase, RPS)],
                    out_hbm.at[c, pl.ds(rbase, RPS)])

  return seg_sum(h, src, dst, zeros)


def _tc_combine(x, maskf, W, b, p0, p1):
  BLK = 2000

  def body(x_ref, m_ref, w_ref, b_ref, p0_ref, p1_ref, o_ref):
    hin = (lax.dot_general(x_ref[...], w_ref[...], (((1,), (1,)), ((), ())),
                           preferred_element_type=jnp.float32)
           + b_ref[...]) * m_ref[...]
    o_ref[...] = jnp.tanh(hin + p0_ref[...] + p1_ref[...])

  return pl.pallas_call(
      body,
      grid=(N // BLK,),
      in_specs=[
          pl.BlockSpec((BLK, D), lambda i: (i, 0)),
          pl.BlockSpec((BLK, 1), lambda i: (i, 0)),
          pl.BlockSpec((D, D), lambda i: (0, 0)),
          pl.BlockSpec((1, D), lambda i: (0, 0)),
          pl.BlockSpec((BLK, D), lambda i: (i, 0)),
          pl.BlockSpec((BLK, D), lambda i: (i, 0)),
      ],
      out_specs=pl.BlockSpec((BLK, D), lambda i: (i, 0)),
      out_shape=jax.ShapeDtypeStruct((N, D), jnp.float32),
  )(x, maskf, W, b, p0, p1)


def kernel(x, h, mask, edge_index, W_in, b_in):
  src = edge_index[0].astype(jnp.int32).reshape(NW, EPW)
  dst = edge_index[1].astype(jnp.int32).reshape(NW, EPW)
  w_ids = jnp.arange(NW, dtype=jnp.int32)[:, None]
  p_ids = jnp.arange(PADW, dtype=jnp.int32)[None, :]
  pad_src = (p_ids * 37 + w_ids * 131) % N
  pad_dst = N + (p_ids + w_ids * 7) % (NPAD - N)
  src = jnp.concatenate([src, pad_src], axis=1).reshape(NW * EPP)
  dst = jnp.concatenate([dst, pad_dst], axis=1).reshape(NW * EPP)
  zeros = jnp.zeros((NPAD, D), jnp.float32)
  partials = _sc_segment_sum(h, src, dst, zeros)
  maskf = mask.astype(jnp.float32)[:, None]
  return _tc_combine(x, maskf, W_in, b_in.reshape(1, D),
                     partials[0, :N], partials[1, :N])


# speedup vs baseline: 13.5388x; 1.1297x over previous
import functools

import jax
import jax.numpy as jnp
from jax import lax
from jax.experimental import pallas as pl
from jax.experimental.pallas import tpu as pltpu
from jax.experimental.pallas import tpu_sc as plsc

N = 10000
E = 320000
D = 128

NC = 2
NS = 16
NW = NC * NS
EPW = E // NW
CHUNK = 80
NCHUNK = EPW // CHUNK
NPAD = 10240
RPS = NPAD // NS
NROW = 3
NSS = 4
NDS = 6


def _sc_segment_sum(h, src, dst, zeros):
  mesh = plsc.VectorSubcoreMesh(core_axis_name="c", subcore_axis_name="s")

  @functools.partial(
      pl.kernel,
      out_type=jax.ShapeDtypeStruct((NC, NPAD, D), jnp.float32),
      mesh=mesh,
      scratch_types=(
          [pltpu.VMEM_SHARED((NPAD, D), jnp.float32)]
          + [pltpu.VMEM((CHUNK, D), jnp.float32)] * NROW
          + [pltpu.VMEM((CHUNK,), jnp.int32)] * NSS
          + [pltpu.VMEM((CHUNK,), jnp.int32)] * NDS
          + [pltpu.SemaphoreType.DMA] * (2 * NROW + NSS + NDS)
      ),
  )
  def seg_sum(h_hbm, src_hbm, dst_hbm, z_hbm, out_hbm, acc, *scr):
    rows = scr[:NROW]
    sslot = scr[NROW:NROW + NSS]
    dslot = scr[NROW + NSS:NROW + NSS + NDS]
    sems = scr[NROW + NSS + NDS:]
    gsem = sems[:NROW]
    ssem = sems[NROW:2 * NROW]
    isem = sems[2 * NROW:2 * NROW + NSS]
    dsem = sems[2 * NROW + NSS:]

    c = lax.axis_index("c")
    s = lax.axis_index("s")
    wid = c * NS + s
    ebase = wid * EPW
    rbase = s * RPS

    def fetch_src(i, q):
      pltpu.async_copy(src_hbm.at[pl.ds(ebase + i * CHUNK, CHUNK)],
                       sslot[q], isem[q])

    def fetch_dst(i, q):
      pltpu.async_copy(dst_hbm.at[pl.ds(ebase + i * CHUNK, CHUNK)],
                       dslot[q], dsem[q])

    def wait_idx(slot, sem):
      pltpu.make_async_copy(src_hbm.at[pl.ds(ebase, CHUNK)], slot, sem).wait()

    def gather(q, b):
      pltpu.async_copy(h_hbm.at[sslot[q]], rows[b], gsem[b])

    def wait_sem(b, sem_ring):
      pltpu.make_async_copy(h_hbm.at[sslot[0]], rows[b], sem_ring[b]).wait()

    def step(i, im, a_on=True, fs_on=True, b_on=True, c_on=True, d_on=True):
      b, q4, q6 = im % NROW, im % NSS, im % NDS
      if a_on:
        wait_sem(b, gsem)
        if fs_on:
          fetch_src(i + 4, q4)
        wait_idx(dslot[q6], dsem[q6])
        pltpu.async_copy(rows[b], acc.at[dslot[q6]], ssem[b], add=True)
      if b_on:
        wait_sem((im + 2) % NROW, ssem)
      if c_on:
        fetch_dst(i + 5, (im + 5) % NDS)
      if d_on:
        wait_idx(sslot[(q4 + 2) % NSS], isem[(q4 + 2) % NSS])
        gather((q4 + 2) % NSS, (im + 2) % NROW)

    pltpu.sync_copy(z_hbm.at[pl.ds(rbase, RPS)], acc.at[pl.ds(rbase, RPS)])
    for q in range(NSS):
      fetch_src(q, q)
    for q in range(NDS - 1):
      fetch_dst(q, q)
    plsc.subcore_barrier()
    for b in range(2):
      wait_idx(sslot[b], isem[b])
      gather(b, b)

    step(0, 0, b_on=False)
    step(1, 1)

    def outer(io, carry):
      for k in range(12):
        step(12 * io + 2 + k, 2 + k)
      return carry

    lax.fori_loop(0, 9, outer, 0)
    for i in range(110, 126):
      step(i, i % 12,
           a_on=(i <= 124),
           fs_on=(i <= 120),
           c_on=(i <= 119),
           d_on=(i <= 122))

    plsc.subcore_barrier()
    pltpu.sync_copy(acc.at[pl.ds(rbase, RPS)],
                    out_hbm.at[c, pl.ds(rbase, RPS)])

  return seg_sum(h, src, dst, zeros)


def _tc_combine(x, maskf, W, b, partials):
  BLK = 2000

  def body(x_ref, m_ref, w_ref, b_ref, p0_ref, p1_ref, o_ref):
    hin = (lax.dot_general(x_ref[...], w_ref[...], (((1,), (1,)), ((), ())),
                           preferred_element_type=jnp.float32)
           + b_ref[...]) * m_ref[...]
    o_ref[...] = jnp.tanh(hin + p0_ref[0] + p1_ref[0])

  return pl.pallas_call(
      body,
      grid=(N // BLK,),
      in_specs=[
          pl.BlockSpec((BLK, D), lambda i: (i, 0)),
          pl.BlockSpec((BLK, 1), lambda i: (i, 0)),
          pl.BlockSpec((D, D), lambda i: (0, 0)),
          pl.BlockSpec((1, D), lambda i: (0, 0)),
          pl.BlockSpec((1, BLK, D), lambda i: (0, i, 0)),
          pl.BlockSpec((1, BLK, D), lambda i: (1, i, 0)),
      ],
      out_specs=pl.BlockSpec((BLK, D), lambda i: (i, 0)),
      out_shape=jax.ShapeDtypeStruct((N, D), jnp.float32),
  )(x, maskf, W, b, partials, partials)


def kernel(x, h, mask, edge_index, W_in, b_in):
  src = edge_index[0].astype(jnp.int32)
  dst = edge_index[1].astype(jnp.int32)
  zeros = jnp.zeros((NPAD, D), jnp.float32)
  partials = _sc_segment_sum(h, src, dst, zeros)
  maskf = mask.astype(jnp.float32)[:, None]
  return _tc_combine(x, maskf, W_in, b_in.reshape(1, D), partials)


# speedup vs baseline: 14.0165x; 1.0353x over previous
import functools

import jax
import jax.numpy as jnp
from jax import lax
from jax.experimental import pallas as pl
from jax.experimental.pallas import tpu as pltpu
from jax.experimental.pallas import tpu_sc as plsc

N = 10000
E = 320000
D = 128

NC = 2
NS = 16
NW = NC * NS
EPW = E // NW
CHUNK = 80
NCHUNK = EPW // CHUNK
NPAD = 10240
RPS = NPAD // NS
NROW = 3
NSS = 4
NDS = 6


def _sc_segment_sum(h, src, dst):
  mesh = plsc.VectorSubcoreMesh(core_axis_name="c", subcore_axis_name="s")

  @functools.partial(
      pl.kernel,
      out_type=jax.ShapeDtypeStruct((NC, NPAD, D), jnp.float32),
      mesh=mesh,
      scratch_types=(
          [pltpu.VMEM_SHARED((NPAD, D), jnp.float32)]
          + [pltpu.VMEM((CHUNK, D), jnp.float32)] * NROW
          + [pltpu.VMEM((CHUNK,), jnp.int32)] * NSS
          + [pltpu.VMEM((CHUNK,), jnp.int32)] * NDS
          + [pltpu.SemaphoreType.DMA] * (2 * NROW + NSS + NDS)
      ),
  )
  def seg_sum(h_hbm, src_hbm, dst_hbm, out_hbm, acc, *scr):
    rows = scr[:NROW]
    sslot = scr[NROW:NROW + NSS]
    dslot = scr[NROW + NSS:NROW + NSS + NDS]
    sems = scr[NROW + NSS + NDS:]
    gsem = sems[:NROW]
    ssem = sems[NROW:2 * NROW]
    isem = sems[2 * NROW:2 * NROW + NSS]
    dsem = sems[2 * NROW + NSS:]

    c = lax.axis_index("c")
    s = lax.axis_index("s")
    wid = c * NS + s
    ebase = wid * EPW
    rbase = s * RPS

    def fetch_src(i, q):
      pltpu.async_copy(src_hbm.at[pl.ds(ebase + i * CHUNK, CHUNK)],
                       sslot[q], isem[q])

    def fetch_dst(i, q):
      pltpu.async_copy(dst_hbm.at[pl.ds(ebase + i * CHUNK, CHUNK)],
                       dslot[q], dsem[q])

    def wait_idx(slot, sem):
      pltpu.make_async_copy(src_hbm.at[pl.ds(ebase, CHUNK)], slot, sem).wait()

    def gather(q, b):
      pltpu.async_copy(h_hbm.at[sslot[q]], rows[b], gsem[b])

    def wait_sem(b, sem_ring):
      pltpu.make_async_copy(h_hbm.at[sslot[0]], rows[b], sem_ring[b]).wait()

    def step(i, im, a_on=True, fs_on=True, b_on=True, c_on=True, d_on=True):
      b, q4, q6 = im % NROW, im % NSS, im % NDS
      if a_on:
        wait_sem(b, gsem)
        if fs_on:
          fetch_src(i + 4, q4)
        wait_idx(dslot[q6], dsem[q6])
        pltpu.async_copy(rows[b], acc.at[dslot[q6]], ssem[b], add=True)
      if b_on:
        wait_sem((im + 2) % NROW, ssem)
      if c_on:
        fetch_dst(i + 5, (im + 5) % NDS)
      if d_on:
        wait_idx(sslot[(q4 + 2) % NSS], isem[(q4 + 2) % NSS])
        gather((q4 + 2) % NSS, (im + 2) % NROW)

    zv = jnp.zeros((16,), jnp.float32)

    def zrow(r, carry):
      for j in range(D // 16):
        rows[0][r, pl.ds(j * 16, 16)] = zv
      return carry

    lax.fori_loop(0, CHUNK, zrow, 0)
    for r2 in range(RPS // CHUNK):
      pltpu.sync_copy(rows[0], acc.at[pl.ds(rbase + r2 * CHUNK, CHUNK)])
    for q in range(NSS):
      fetch_src(q, q)
    for q in range(NDS - 1):
      fetch_dst(q, q)
    plsc.subcore_barrier()
    for b in range(2):
      wait_idx(sslot[b], isem[b])
      gather(b, b)

    step(0, 0, b_on=False)
    step(1, 1)

    def outer(io, carry):
      for k in range(12):
        step(12 * io + 2 + k, 2 + k)
      return carry

    lax.fori_loop(0, 9, outer, 0)
    for i in range(110, 126):
      step(i, i % 12,
           a_on=(i <= 124),
           fs_on=(i <= 120),
           c_on=(i <= 119),
           d_on=(i <= 122))

    plsc.subcore_barrier()
    pltpu.sync_copy(acc.at[pl.ds(rbase, RPS)],
                    out_hbm.at[c, pl.ds(rbase, RPS)])

  return seg_sum(h, src, dst)


def _tc_combine(x, maskf, W, b, partials):
  BLK = 2000

  def body(x_ref, m_ref, w_ref, b_ref, p0_ref, p1_ref, o_ref):
    hin = (lax.dot_general(x_ref[...], w_ref[...], (((1,), (1,)), ((), ())),
                           preferred_element_type=jnp.float32)
           + b_ref[...]) * m_ref[...]
    o_ref[...] = jnp.tanh(hin + p0_ref[0] + p1_ref[0])

  return pl.pallas_call(
      body,
      grid=(N // BLK,),
      in_specs=[
          pl.BlockSpec((BLK, D), lambda i: (i, 0)),
          pl.BlockSpec((BLK, 1), lambda i: (i, 0)),
          pl.BlockSpec((D, D), lambda i: (0, 0)),
          pl.BlockSpec((1, D), lambda i: (0, 0)),
          pl.BlockSpec((1, BLK, D), lambda i: (0, i, 0)),
          pl.BlockSpec((1, BLK, D), lambda i: (1, i, 0)),
      ],
      out_specs=pl.BlockSpec((BLK, D), lambda i: (i, 0)),
      out_shape=jax.ShapeDtypeStruct((N, D), jnp.float32),
  )(x, maskf, W, b, partials, partials)


def kernel(x, h, mask, edge_index, W_in, b_in):
  src = edge_index[0].astype(jnp.int32)
  dst = edge_index[1].astype(jnp.int32)
  partials = _sc_segment_sum(h, src, dst)
  maskf = mask.astype(jnp.float32)[:, None]
  return _tc_combine(x, maskf, W_in, b_in.reshape(1, D), partials)
